# Initial kernel scaffold; baseline (speedup 1.0000x reference)
import jax
import jax.numpy as jnp
from jax.experimental import pallas as pl


def kernel(words, ext_words, tags, W_word, W_ext, W_tag):
    raise NotImplementedError("write your pallas kernel here")


# speedup vs baseline: 2.3234x; 2.3234x over previous
import jax
import jax.numpy as jnp
from jax import lax
from jax.experimental import pallas as pl
from jax.experimental.pallas import tpu as pltpu
from jax.experimental.pallas import tpu_sc as plsc

B, L = 4096, 50
WORD_DIM, TAG_DIM = 128, 64
OUT_DIM = WORD_DIM + TAG_DIM
N = B * L
NC, NS = 2, 16
NW = NC * NS
C = 128
CHUNKS_PER_W = N // (NW * C)


def _sc_body(words_hbm, ext_hbm, tags_hbm, w_word_hbm, w_ext_hbm, w_tag_hbm,
             out_hbm, idx_w, idx_e, idx_t, buf_w, buf_e, buf_t, out_buf,
             sem_w, sem_e, sem_t):
    wid = lax.axis_index("s") * NC + lax.axis_index("c")
    row0 = wid * CHUNKS_PER_W

    pltpu.sync_copy(words_hbm.at[wid], idx_w)
    pltpu.sync_copy(ext_hbm.at[wid], idx_e)
    pltpu.sync_copy(tags_hbm.at[wid], idx_t)

    def chunk(j, carry):
        base = (row0 + j) * C
        cw = pltpu.async_copy(w_word_hbm.at[idx_w.at[j]], buf_w, sem_w)
        ce = pltpu.async_copy(w_ext_hbm.at[idx_e.at[j]], buf_e, sem_e)
        ct = pltpu.async_copy(w_tag_hbm.at[idx_t.at[j]], buf_t, sem_t)
        cw.wait()
        ce.wait()
        ct.wait()

        def assemble_row(r, c2):
            for cc in range(WORD_DIM // 16):
                s = pl.ds(cc * 16, 16)
                out_buf[r, s] = buf_w[r, s] + buf_e[r, s]
            for cc in range(TAG_DIM // 16):
                out_buf[r, pl.ds(WORD_DIM + cc * 16, 16)] = \
                    buf_t[r, pl.ds(cc * 16, 16)]
            return c2

        lax.fori_loop(0, C, assemble_row, 0)
        pltpu.sync_copy(out_buf, out_hbm.at[pl.ds(base, C)])
        return carry

    lax.fori_loop(0, CHUNKS_PER_W, chunk, 0)


@jax.jit
def _input_layer(words, ext_words, tags, w_word, w_ext, w_tag):
    words2 = words.reshape(-1).astype(jnp.int32).reshape(NW, CHUNKS_PER_W, C)
    ext2 = ext_words.reshape(-1).astype(jnp.int32).reshape(NW, CHUNKS_PER_W, C)
    tags2 = tags.reshape(-1).astype(jnp.int32).reshape(NW, CHUNKS_PER_W, C)
    w_tag_pad = jnp.pad(w_tag, ((0, 0), (0, WORD_DIM - TAG_DIM)))

    mesh = plsc.VectorSubcoreMesh(core_axis_name="c", subcore_axis_name="s",
                                  num_cores=NC, num_subcores=NS)
    out = pl.kernel(
        _sc_body,
        out_type=jax.ShapeDtypeStruct((N, OUT_DIM), jnp.float32),
        mesh=mesh,
        scratch_types=[
            pltpu.VMEM((CHUNKS_PER_W, C), jnp.int32),
            pltpu.VMEM((CHUNKS_PER_W, C), jnp.int32),
            pltpu.VMEM((CHUNKS_PER_W, C), jnp.int32),
            pltpu.VMEM((C, WORD_DIM), jnp.float32),
            pltpu.VMEM((C, WORD_DIM), jnp.float32),
            pltpu.VMEM((C, WORD_DIM), jnp.float32),
            pltpu.VMEM((C, OUT_DIM), jnp.float32),
            pltpu.SemaphoreType.DMA,
            pltpu.SemaphoreType.DMA,
            pltpu.SemaphoreType.DMA,
        ],
    )(words2, ext2, tags2, w_word, w_ext, w_tag_pad)
    return out.reshape(B, L, OUT_DIM)


def kernel(words, ext_words, tags, W_word, W_ext, W_tag):
    return _input_layer(words, ext_words, tags, W_word, W_ext, W_tag)


# speedup vs baseline: 2.9489x; 1.2692x over previous
import jax
import jax.numpy as jnp
from jax import lax
from jax.experimental import pallas as pl
from jax.experimental.pallas import tpu as pltpu
from jax.experimental.pallas import tpu_sc as plsc

B, L = 4096, 50
WORD_DIM, TAG_DIM = 128, 64
OUT_DIM = WORD_DIM + TAG_DIM
N = B * L
NC, NS = 2, 16
NW = NC * NS
C = 64
CHUNKS_PER_W = N // (NW * C)


def _sc_body(words_hbm, ext_hbm, tags_hbm, w_word_hbm, w_ext_hbm, w_tag_hbm,
             out_hbm,
             idx_w, idx_e, idx_t,
             buf_w0, buf_e0, buf_t0, out0,
             buf_w1, buf_e1, buf_t1, out1,
             sem_g0, sem_g1, sem_o0, sem_o1):
    wid = lax.axis_index("s") * NC + lax.axis_index("c")
    row0 = wid * CHUNKS_PER_W

    pltpu.sync_copy(words_hbm.at[wid], idx_w)
    pltpu.sync_copy(ext_hbm.at[wid], idx_e)
    pltpu.sync_copy(tags_hbm.at[wid], idx_t)

    sets = ((buf_w0, buf_e0, buf_t0, out0, sem_g0, sem_o0),
            (buf_w1, buf_e1, buf_t1, out1, sem_g1, sem_o1))

    def issue(j, s):
        bw, be, bt, _, sem, _ = s
        pltpu.async_copy(w_word_hbm.at[idx_w.at[j]], bw, sem)
        pltpu.async_copy(w_ext_hbm.at[idx_e.at[j]], be, sem)
        pltpu.async_copy(w_tag_hbm.at[idx_t.at[j]], bt, sem)

    def drain_gathers(j, s):
        bw, be, bt, _, sem, _ = s
        pltpu.make_async_copy(w_word_hbm.at[idx_w.at[j]], bw, sem).wait()
        pltpu.make_async_copy(w_ext_hbm.at[idx_e.at[j]], be, sem).wait()
        pltpu.make_async_copy(w_tag_hbm.at[idx_t.at[j]], bt, sem).wait()

    def assemble_and_write(j, s, first):
        bw, be, bt, ob, _, sem_o = s
        drain_gathers(j, s)
        if not first:
            pltpu.make_async_copy(ob, out_hbm.at[pl.ds(0, C)], sem_o).wait()

        def assemble_row(r, c2):
            for cc in range(WORD_DIM // 16):
                sl = pl.ds(cc * 16, 16)
                ob[r, sl] = bw[r, sl] + be[r, sl]
            for cc in range(TAG_DIM // 16):
                ob[r, pl.ds(WORD_DIM + cc * 16, 16)] = bt[r, pl.ds(cc * 16, 16)]
            return c2

        lax.fori_loop(0, C, assemble_row, 0)
        base = (row0 + j) * C
        pltpu.async_copy(ob, out_hbm.at[pl.ds(base, C)], sem_o)

    issue(0, sets[0])
    issue(1, sets[1])
    assemble_and_write(0, sets[0], first=True)
    issue(2, sets[0])
    assemble_and_write(1, sets[1], first=True)
    issue(3, sets[1])

    def body(jj, carry):
        j = jj * 2
        assemble_and_write(j, sets[0], first=False)

        @pl.when(j + 2 < CHUNKS_PER_W)
        def _():
            issue(j + 2, sets[0])

        assemble_and_write(j + 1, sets[1], first=False)

        @pl.when(j + 3 < CHUNKS_PER_W)
        def _():
            issue(j + 3, sets[1])

        return carry

    lax.fori_loop(1, CHUNKS_PER_W // 2, body, 0)

    pltpu.make_async_copy(out0, out_hbm.at[pl.ds(0, C)], sem_o0).wait()
    pltpu.make_async_copy(out1, out_hbm.at[pl.ds(0, C)], sem_o1).wait()


@jax.jit
def _input_layer(words, ext_words, tags, w_word, w_ext, w_tag):
    words2 = words.reshape(-1).astype(jnp.int32).reshape(NW, CHUNKS_PER_W, C)
    ext2 = ext_words.reshape(-1).astype(jnp.int32).reshape(NW, CHUNKS_PER_W, C)
    tags2 = tags.reshape(-1).astype(jnp.int32).reshape(NW, CHUNKS_PER_W, C)
    w_tag_pad = jnp.pad(w_tag, ((0, 0), (0, WORD_DIM - TAG_DIM)))

    mesh = plsc.VectorSubcoreMesh(core_axis_name="c", subcore_axis_name="s",
                                  num_cores=NC, num_subcores=NS)
    out = pl.kernel(
        _sc_body,
        out_type=jax.ShapeDtypeStruct((N, OUT_DIM), jnp.float32),
        mesh=mesh,
        scratch_types=[
            pltpu.VMEM((CHUNKS_PER_W, C), jnp.int32),
            pltpu.VMEM((CHUNKS_PER_W, C), jnp.int32),
            pltpu.VMEM((CHUNKS_PER_W, C), jnp.int32),
            pltpu.VMEM((C, WORD_DIM), jnp.float32),
            pltpu.VMEM((C, WORD_DIM), jnp.float32),
            pltpu.VMEM((C, WORD_DIM), jnp.float32),
            pltpu.VMEM((C, OUT_DIM), jnp.float32),
            pltpu.VMEM((C, WORD_DIM), jnp.float32),
            pltpu.VMEM((C, WORD_DIM), jnp.float32),
            pltpu.VMEM((C, WORD_DIM), jnp.float32),
            pltpu.VMEM((C, OUT_DIM), jnp.float32),
            pltpu.SemaphoreType.DMA,
            pltpu.SemaphoreType.DMA,
            pltpu.SemaphoreType.DMA,
            pltpu.SemaphoreType.DMA,
        ],
    )(words2, ext2, tags2, w_word, w_ext, w_tag_pad)
    return out.reshape(B, L, OUT_DIM)


def kernel(words, ext_words, tags, W_word, W_ext, W_tag):
    return _input_layer(words, ext_words, tags, W_word, W_ext, W_tag)
